# baseline (speedup 1.0000x reference)
import functools

import jax
import jax.numpy as jnp
from jax import lax
from jax.experimental import pallas as pl
from jax.experimental.pallas import tpu as pltpu
from jax.experimental.pallas import tpu_sc as plsc

N_NODES = 10000
N_EDGES = 320000
D_FEAT = 128

NC = 2
NS = 16
E_PER_TILE = N_EDGES // (NC * NS)
CHUNK = 125
NGROUP = 10
GCHUNK = 8
NCHUNK = NGROUP * GCHUNK
NBLK = NGROUP // 2
TW = 632
ZC = 96

_mesh = plsc.VectorSubcoreMesh(core_axis_name="c", subcore_axis_name="s")


@jax.jit
def _propagate(x, ei5):
    @functools.partial(
        pl.kernel,
        out_type=jax.ShapeDtypeStruct((N_NODES, NC * D_FEAT), jnp.float32),
        mesh=_mesh,
        scratch_types=[
            pltpu.VMEM((GCHUNK, CHUNK), jnp.int32),
            pltpu.VMEM((GCHUNK, CHUNK), jnp.int32),
            pltpu.VMEM((GCHUNK, CHUNK), jnp.int32),
            pltpu.VMEM((GCHUNK, CHUNK), jnp.int32),
            pltpu.VMEM((2, CHUNK, D_FEAT), jnp.float32),
            pltpu.VMEM_SHARED((N_NODES, D_FEAT), jnp.float32),
            pltpu.SemaphoreType.DMA,
            pltpu.SemaphoreType.DMA,
            pltpu.SemaphoreType.DMA,
            pltpu.SemaphoreType.DMA,
        ],
    )
    def sc_kernel(x_hbm, ei_hbm, out_hbm,
                  src_v0, dst_v0, src_v1, dst_v1, rows_v, acc,
                  isem0, isem1, gsem0, gsem1):
        c = lax.axis_index("c")
        s = lax.axis_index("s")

        idx_sets = ((src_v0, dst_v0, isem0), (src_v1, dst_v1, isem1))
        row_bufs = ((rows_v.at[0], gsem0), (rows_v.at[1], gsem1))

        def grows(g):
            return pl.ds(pl.multiple_of(g * GCHUNK, 8), GCHUNK)

        def stage(g, set_id):
            src_b, dst_b, isem = idx_sets[set_id]
            pltpu.async_copy(ei_hbm.at[0, c, s, grows(g)], src_b, isem)
            pltpu.async_copy(ei_hbm.at[1, c, s, grows(g)], dst_b, isem)

        def stage_wait(g, set_id):
            src_b, dst_b, isem = idx_sets[set_id]
            pltpu.make_async_copy(
                ei_hbm.at[0, c, s, grows(g)], src_b, isem).wait()
            pltpu.make_async_copy(
                ei_hbm.at[1, c, s, grows(g)], dst_b, isem).wait()

        def gather_start(set_id, r, buf_id):
            src_b = idx_sets[set_id][0]
            buf, gsem = row_bufs[buf_id]
            pltpu.async_copy(x_hbm.at[src_b.at[r]], buf, gsem)

        def gather_wait(set_id, r, buf_id):
            src_b = idx_sets[set_id][0]
            buf, gsem = row_bufs[buf_id]
            pltpu.make_async_copy(x_hbm.at[src_b.at[r]], buf, gsem).wait()

        stage(0, 0)
        stage(1, 1)

        zeros16 = jnp.zeros((16,), jnp.float32)

        @pl.loop(0, ZC)
        def _(i):
            @pl.loop(0, D_FEAT, step=16)
            def _(k):
                rows_v[1, i, pl.ds(k, 16)] = zeros16

        stage_wait(0, 0)
        gather_start(0, 0, 0)

        start = pl.multiple_of(jnp.minimum(s * TW, N_NODES - TW), 8)

        @pl.loop(0, TW // ZC)
        def _(k):
            pltpu.sync_copy(
                rows_v.at[1, pl.ds(0, ZC)],
                acc.at[pl.ds(pl.multiple_of(start + k * ZC, 8), ZC)])

        rem = TW - (TW // ZC) * ZC
        pltpu.sync_copy(
            rows_v.at[1, pl.ds(0, rem)],
            acc.at[pl.ds(pl.multiple_of(start + TW - rem, 8), rem)])

        plsc.subcore_barrier()

        @pl.loop(0, NBLK)
        def _(b):
            g0 = b * 2
            g1 = g0 + 1

            for k in range(2 * GCHUNK):
                set_id = 0 if k < GCHUNK else 1
                r = k % GCHUNK
                buf_id = k % 2

                if k == 0:
                    @pl.when(b > 0)
                    def _():
                        stage(g1, 1)
                if k == GCHUNK:
                    @pl.when(g0 + 2 < NGROUP)
                    def _():
                        stage(g0 + 2, 0)

                nk = k + 1
                if nk < 2 * GCHUNK:
                    if nk == GCHUNK:
                        stage_wait(g1, 1)
                    gather_start(0 if nk < GCHUNK else 1, nk % GCHUNK,
                                 nk % 2)
                else:
                    @pl.when(g0 + 2 < NGROUP)
                    def _():
                        stage_wait(g0 + 2, 0)
                        gather_start(0, 0, 0)

                dst_b = idx_sets[set_id][1]
                gather_wait(set_id, r, buf_id)
                pltpu.sync_copy(rows_v.at[buf_id], acc.at[dst_b.at[r]],
                                add=True)

        plsc.subcore_barrier()

        pltpu.sync_copy(
            acc.at[pl.ds(start, TW)],
            out_hbm.at[pl.ds(start, TW),
                       pl.ds(pl.multiple_of(c * D_FEAT, 128), D_FEAT)],
        )

    return sc_kernel(x, ei5)


def _add_body(p_ref, o_ref):
    o_ref[...] = p_ref[:, :D_FEAT] + p_ref[:, D_FEAT:]


@jax.jit
def _combine(partials):
    return pl.pallas_call(
        _add_body,
        out_shape=jax.ShapeDtypeStruct((N_NODES, D_FEAT), jnp.float32),
        grid=(10,),
        in_specs=[
            pl.BlockSpec((N_NODES // 10, NC * D_FEAT), lambda i: (i, 0)),
        ],
        out_specs=pl.BlockSpec((N_NODES // 10, D_FEAT), lambda i: (i, 0)),
    )(partials)


def kernel(x, edge_index):
    ei5 = edge_index.reshape(2, NC, NS, NCHUNK, CHUNK)
    partials = _propagate(x, ei5)
    return _combine(partials)


# speedup vs baseline: 1.0025x; 1.0025x over previous
import functools

import jax
import jax.numpy as jnp
from jax import lax
from jax.experimental import pallas as pl
from jax.experimental.pallas import tpu as pltpu
from jax.experimental.pallas import tpu_sc as plsc

N_NODES = 10000
N_EDGES = 320000
D_FEAT = 128

NC = 2
NS = 16
E_PER_TILE = N_EDGES // (NC * NS)
CHUNK = 125
NGROUP = 10
GCHUNK = 8
NCHUNK = NGROUP * GCHUNK
NBLK = NGROUP // 2
TW = 632
ZC = 96

_mesh = plsc.VectorSubcoreMesh(core_axis_name="c", subcore_axis_name="s")


@jax.jit
def _propagate(x, ei5):
    @functools.partial(
        pl.kernel,
        out_type=jax.ShapeDtypeStruct((NC, N_NODES, D_FEAT), jnp.float32),
        mesh=_mesh,
        scratch_types=[
            pltpu.VMEM((GCHUNK, CHUNK), jnp.int32),
            pltpu.VMEM((GCHUNK, CHUNK), jnp.int32),
            pltpu.VMEM((GCHUNK, CHUNK), jnp.int32),
            pltpu.VMEM((GCHUNK, CHUNK), jnp.int32),
            pltpu.VMEM((2, CHUNK, D_FEAT), jnp.float32),
            pltpu.VMEM_SHARED((N_NODES, D_FEAT), jnp.float32),
            pltpu.SemaphoreType.DMA,
            pltpu.SemaphoreType.DMA,
            pltpu.SemaphoreType.DMA,
            pltpu.SemaphoreType.DMA,
        ],
    )
    def sc_kernel(x_hbm, ei_hbm, out_hbm,
                  src_v0, dst_v0, src_v1, dst_v1, rows_v, acc,
                  isem0, isem1, gsem0, gsem1):
        c = lax.axis_index("c")
        s = lax.axis_index("s")

        idx_sets = ((src_v0, dst_v0, isem0), (src_v1, dst_v1, isem1))
        row_bufs = ((rows_v.at[0], gsem0), (rows_v.at[1], gsem1))

        def grows(g):
            return pl.ds(pl.multiple_of(g * GCHUNK, 8), GCHUNK)

        def stage(g, set_id):
            src_b, dst_b, isem = idx_sets[set_id]
            pltpu.async_copy(ei_hbm.at[0, c, s, grows(g)], src_b, isem)
            pltpu.async_copy(ei_hbm.at[1, c, s, grows(g)], dst_b, isem)

        def stage_wait(g, set_id):
            src_b, dst_b, isem = idx_sets[set_id]
            pltpu.make_async_copy(
                ei_hbm.at[0, c, s, grows(g)], src_b, isem).wait()
            pltpu.make_async_copy(
                ei_hbm.at[1, c, s, grows(g)], dst_b, isem).wait()

        def gather_start(set_id, r, buf_id):
            src_b = idx_sets[set_id][0]
            buf, gsem = row_bufs[buf_id]
            pltpu.async_copy(x_hbm.at[src_b.at[r]], buf, gsem)

        def gather_wait(set_id, r, buf_id):
            src_b = idx_sets[set_id][0]
            buf, gsem = row_bufs[buf_id]
            pltpu.make_async_copy(x_hbm.at[src_b.at[r]], buf, gsem).wait()

        stage(0, 0)
        stage(1, 1)

        zeros16 = jnp.zeros((16,), jnp.float32)

        @pl.loop(0, ZC)
        def _(i):
            @pl.loop(0, D_FEAT, step=16)
            def _(k):
                rows_v[1, i, pl.ds(k, 16)] = zeros16

        stage_wait(0, 0)
        gather_start(0, 0, 0)

        start = pl.multiple_of(jnp.minimum(s * TW, N_NODES - TW), 8)

        @pl.loop(0, TW // ZC)
        def _(k):
            pltpu.sync_copy(
                rows_v.at[1, pl.ds(0, ZC)],
                acc.at[pl.ds(pl.multiple_of(start + k * ZC, 8), ZC)])

        rem = TW - (TW // ZC) * ZC
        pltpu.sync_copy(
            rows_v.at[1, pl.ds(0, rem)],
            acc.at[pl.ds(pl.multiple_of(start + TW - rem, 8), rem)])

        plsc.subcore_barrier()

        @pl.loop(0, NBLK)
        def _(b):
            g0 = b * 2
            g1 = g0 + 1

            for k in range(2 * GCHUNK):
                set_id = 0 if k < GCHUNK else 1
                r = k % GCHUNK
                buf_id = k % 2

                if k == 0:
                    @pl.when(b > 0)
                    def _():
                        stage(g1, 1)
                if k == GCHUNK:
                    @pl.when(g0 + 2 < NGROUP)
                    def _():
                        stage(g0 + 2, 0)

                nk = k + 1
                if nk < 2 * GCHUNK:
                    if nk == GCHUNK:
                        stage_wait(g1, 1)
                    gather_start(0 if nk < GCHUNK else 1, nk % GCHUNK,
                                 nk % 2)
                else:
                    @pl.when(g0 + 2 < NGROUP)
                    def _():
                        stage_wait(g0 + 2, 0)
                        gather_start(0, 0, 0)

                dst_b = idx_sets[set_id][1]
                gather_wait(set_id, r, buf_id)
                pltpu.sync_copy(rows_v.at[buf_id], acc.at[dst_b.at[r]],
                                add=True)

        plsc.subcore_barrier()

        pltpu.sync_copy(
            acc.at[pl.ds(start, TW)],
            out_hbm.at[c, pl.ds(start, TW)],
        )

    return sc_kernel(x, ei5)


def _add_body(p_ref, o_ref):
    o_ref[...] = p_ref[0] + p_ref[1]


@jax.jit
def _combine(partials):
    return pl.pallas_call(
        _add_body,
        out_shape=jax.ShapeDtypeStruct((N_NODES, D_FEAT), jnp.float32),
        grid=(10,),
        in_specs=[
            pl.BlockSpec((NC, N_NODES // 10, D_FEAT), lambda i: (0, i, 0)),
        ],
        out_specs=pl.BlockSpec((N_NODES // 10, D_FEAT), lambda i: (i, 0)),
    )(partials)


def kernel(x, edge_index):
    ei5 = edge_index.reshape(2, NC, NS, NCHUNK, CHUNK)
    partials = _propagate(x, ei5)
    return _combine(partials)


# speedup vs baseline: 1.0186x; 1.0161x over previous
import functools

import jax
import jax.numpy as jnp
from jax import lax
from jax.experimental import pallas as pl
from jax.experimental.pallas import tpu as pltpu
from jax.experimental.pallas import tpu_sc as plsc

N_NODES = 10000
N_EDGES = 320000
D_FEAT = 128

NC = 2
NS = 16
E_PER_TILE = N_EDGES // (NC * NS)
CHUNK = 125
NGROUP = 8
GCHUNK = 10
NCHUNK = NGROUP * GCHUNK
NBLK = NGROUP // 2
TW = 632
ZC = 96

_mesh = plsc.VectorSubcoreMesh(core_axis_name="c", subcore_axis_name="s")


@jax.jit
def _propagate(x, ei6):
    @functools.partial(
        pl.kernel,
        out_type=jax.ShapeDtypeStruct((NC, N_NODES, D_FEAT), jnp.float32),
        mesh=_mesh,
        scratch_types=[
            pltpu.VMEM((GCHUNK, CHUNK), jnp.int32),
            pltpu.VMEM((GCHUNK, CHUNK), jnp.int32),
            pltpu.VMEM((GCHUNK, CHUNK), jnp.int32),
            pltpu.VMEM((GCHUNK, CHUNK), jnp.int32),
            pltpu.VMEM((2, CHUNK, D_FEAT), jnp.float32),
            pltpu.VMEM_SHARED((N_NODES, D_FEAT), jnp.float32),
            pltpu.SemaphoreType.DMA,
            pltpu.SemaphoreType.DMA,
            pltpu.SemaphoreType.DMA,
            pltpu.SemaphoreType.DMA,
        ],
    )
    def sc_kernel(x_hbm, ei_hbm, out_hbm,
                  src_v0, dst_v0, src_v1, dst_v1, rows_v, acc,
                  isem0, isem1, gsem0, gsem1):
        c = lax.axis_index("c")
        s = lax.axis_index("s")

        idx_sets = ((src_v0, dst_v0, isem0), (src_v1, dst_v1, isem1))
        row_bufs = ((rows_v.at[0], gsem0), (rows_v.at[1], gsem1))

        def stage(g, set_id):
            src_b, dst_b, isem = idx_sets[set_id]
            pltpu.async_copy(ei_hbm.at[0, c, s, g], src_b, isem)
            pltpu.async_copy(ei_hbm.at[1, c, s, g], dst_b, isem)

        def stage_wait(g, set_id):
            src_b, dst_b, isem = idx_sets[set_id]
            pltpu.make_async_copy(
                ei_hbm.at[0, c, s, g], src_b, isem).wait()
            pltpu.make_async_copy(
                ei_hbm.at[1, c, s, g], dst_b, isem).wait()

        def gather_start(set_id, r, buf_id):
            src_b = idx_sets[set_id][0]
            buf, gsem = row_bufs[buf_id]
            pltpu.async_copy(x_hbm.at[src_b.at[r]], buf, gsem)

        def gather_wait(set_id, r, buf_id):
            src_b = idx_sets[set_id][0]
            buf, gsem = row_bufs[buf_id]
            pltpu.make_async_copy(x_hbm.at[src_b.at[r]], buf, gsem).wait()

        stage(0, 0)
        stage(1, 1)

        zeros16 = jnp.zeros((16,), jnp.float32)

        @pl.loop(0, ZC)
        def _(i):
            @pl.loop(0, D_FEAT, step=16)
            def _(k):
                rows_v[1, i, pl.ds(k, 16)] = zeros16

        stage_wait(0, 0)
        gather_start(0, 0, 0)

        start = pl.multiple_of(jnp.minimum(s * TW, N_NODES - TW), 8)

        @pl.loop(0, TW // ZC)
        def _(k):
            pltpu.sync_copy(
                rows_v.at[1, pl.ds(0, ZC)],
                acc.at[pl.ds(pl.multiple_of(start + k * ZC, 8), ZC)])

        rem = TW - (TW // ZC) * ZC
        pltpu.sync_copy(
            rows_v.at[1, pl.ds(0, rem)],
            acc.at[pl.ds(pl.multiple_of(start + TW - rem, 8), rem)])

        plsc.subcore_barrier()

        @pl.loop(0, NBLK)
        def _(b):
            g0 = b * 2
            g1 = g0 + 1

            for k in range(2 * GCHUNK):
                set_id = 0 if k < GCHUNK else 1
                r = k % GCHUNK
                buf_id = k % 2

                if k == 0:
                    @pl.when(b > 0)
                    def _():
                        stage(g1, 1)
                if k == GCHUNK:
                    @pl.when(g0 + 2 < NGROUP)
                    def _():
                        stage(g0 + 2, 0)

                nk = k + 1
                if nk < 2 * GCHUNK:
                    if nk == GCHUNK:
                        stage_wait(g1, 1)
                    gather_start(0 if nk < GCHUNK else 1, nk % GCHUNK,
                                 nk % 2)
                else:
                    @pl.when(g0 + 2 < NGROUP)
                    def _():
                        stage_wait(g0 + 2, 0)
                        gather_start(0, 0, 0)

                dst_b = idx_sets[set_id][1]
                gather_wait(set_id, r, buf_id)
                pltpu.sync_copy(rows_v.at[buf_id], acc.at[dst_b.at[r]],
                                add=True)

        plsc.subcore_barrier()

        pltpu.sync_copy(
            acc.at[pl.ds(start, TW)],
            out_hbm.at[c, pl.ds(start, TW)],
        )

    return sc_kernel(x, ei6)


def _add_body(p_ref, o_ref):
    o_ref[...] = p_ref[0] + p_ref[1]


@jax.jit
def _combine(partials):
    return pl.pallas_call(
        _add_body,
        out_shape=jax.ShapeDtypeStruct((N_NODES, D_FEAT), jnp.float32),
        grid=(10,),
        in_specs=[
            pl.BlockSpec((NC, N_NODES // 10, D_FEAT), lambda i: (0, i, 0)),
        ],
        out_specs=pl.BlockSpec((N_NODES // 10, D_FEAT), lambda i: (i, 0)),
    )(partials)


def kernel(x, edge_index):
    ei6 = edge_index.reshape(2, NC, NS, NGROUP, GCHUNK, CHUNK)
    partials = _propagate(x, ei6)
    return _combine(partials)


# speedup vs baseline: 1.0309x; 1.0121x over previous
import functools

import jax
import jax.numpy as jnp
from jax import lax
from jax.experimental import pallas as pl
from jax.experimental.pallas import tpu as pltpu
from jax.experimental.pallas import tpu_sc as plsc

N_NODES = 10000
N_EDGES = 320000
D_FEAT = 128

NC = 2
NS = 16
E_PER_TILE = N_EDGES // (NC * NS)
CHUNK = 125
NGROUP = 8
GCHUNK = 10
NCHUNK = NGROUP * GCHUNK
NBLK = NGROUP // 2
TW = 632
ZC = 96

_mesh = plsc.VectorSubcoreMesh(core_axis_name="c", subcore_axis_name="s")


@jax.jit
def _propagate(x, ei6):
    @functools.partial(
        pl.kernel,
        out_type=jax.ShapeDtypeStruct((NC, N_NODES, D_FEAT), jnp.float32),
        mesh=_mesh,
        scratch_types=[
            pltpu.VMEM((GCHUNK, CHUNK), jnp.int32),
            pltpu.VMEM((GCHUNK, CHUNK), jnp.int32),
            pltpu.VMEM((GCHUNK, CHUNK), jnp.int32),
            pltpu.VMEM((GCHUNK, CHUNK), jnp.int32),
            pltpu.VMEM((2, CHUNK, D_FEAT), jnp.float32),
            pltpu.VMEM_SHARED((N_NODES, D_FEAT), jnp.float32),
            pltpu.SemaphoreType.DMA,
            pltpu.SemaphoreType.DMA,
            pltpu.SemaphoreType.DMA,
            pltpu.SemaphoreType.DMA,
        ],
    )
    def sc_kernel(x_hbm, ei_hbm, out_hbm,
                  src_v0, dst_v0, src_v1, dst_v1, rows_v, acc,
                  isem0, isem1, gsem0, gsem1):
        c = lax.axis_index("c")
        s = lax.axis_index("s")

        idx_sets = ((src_v0, dst_v0, isem0), (src_v1, dst_v1, isem1))
        row_bufs = ((rows_v.at[0], gsem0), (rows_v.at[1], gsem1))

        def stage(g, set_id):
            src_b, dst_b, isem = idx_sets[set_id]
            pltpu.async_copy(ei_hbm.at[0, c, s, g], src_b, isem)
            pltpu.async_copy(ei_hbm.at[1, c, s, g], dst_b, isem)

        def stage_wait(g, set_id):
            src_b, dst_b, isem = idx_sets[set_id]
            pltpu.make_async_copy(
                ei_hbm.at[0, c, s, g], src_b, isem).wait()
            pltpu.make_async_copy(
                ei_hbm.at[1, c, s, g], dst_b, isem).wait()

        def gather_start(set_id, r, buf_id):
            src_b = idx_sets[set_id][0]
            buf, gsem = row_bufs[buf_id]
            pltpu.async_copy(x_hbm.at[src_b.at[r]], buf, gsem)

        def gather_wait(set_id, r, buf_id):
            src_b = idx_sets[set_id][0]
            buf, gsem = row_bufs[buf_id]
            pltpu.make_async_copy(x_hbm.at[src_b.at[r]], buf, gsem).wait()

        stage(0, 0)
        stage(1, 1)

        zeros16 = jnp.zeros((16,), jnp.float32)

        @pl.loop(0, ZC)
        def _(i):
            @pl.loop(0, D_FEAT, step=16)
            def _(k):
                rows_v[1, i, pl.ds(k, 16)] = zeros16

        stage_wait(0, 0)
        gather_start(0, 0, 0)

        start = pl.multiple_of(jnp.minimum(s * TW, N_NODES - TW), 8)

        @pl.loop(0, TW // ZC)
        def _(k):
            pltpu.sync_copy(
                rows_v.at[1, pl.ds(0, ZC)],
                acc.at[pl.ds(pl.multiple_of(start + k * ZC, 8), ZC)])

        rem = TW - (TW // ZC) * ZC
        pltpu.sync_copy(
            rows_v.at[1, pl.ds(0, rem)],
            acc.at[pl.ds(pl.multiple_of(start + TW - rem, 8), rem)])

        plsc.subcore_barrier()

        @pl.loop(0, NBLK)
        def _(b):
            g0 = b * 2
            g1 = g0 + 1

            for k in range(2 * GCHUNK):
                set_id = 0 if k < GCHUNK else 1
                r = k % GCHUNK
                buf_id = k % 2

                if k == 0:
                    @pl.when(b > 0)
                    def _():
                        stage(g1, 1)
                if k == GCHUNK:
                    @pl.when(g0 + 2 < NGROUP)
                    def _():
                        stage(g0 + 2, 0)

                nk = k + 1
                if nk < 2 * GCHUNK:
                    if nk == GCHUNK:
                        stage_wait(g1, 1)
                    gather_start(0 if nk < GCHUNK else 1, nk % GCHUNK,
                                 nk % 2)
                else:
                    @pl.when(g0 + 2 < NGROUP)
                    def _():
                        stage_wait(g0 + 2, 0)
                        gather_start(0, 0, 0)

                dst_b = idx_sets[set_id][1]
                gather_wait(set_id, r, buf_id)
                pltpu.sync_copy(rows_v.at[buf_id], acc.at[dst_b.at[r]],
                                add=True)

        plsc.subcore_barrier()

        pltpu.sync_copy(
            acc.at[pl.ds(start, TW)],
            out_hbm.at[c, pl.ds(start, TW)],
        )

    return sc_kernel(x, ei6)


def _add_body(p_ref, o_ref):
    o_ref[...] = p_ref[0] + p_ref[1]


@jax.jit
def _combine(partials):
    return pl.pallas_call(
        _add_body,
        out_shape=jax.ShapeDtypeStruct((N_NODES, D_FEAT), jnp.float32),
        grid=(5,),
        in_specs=[
            pl.BlockSpec((NC, N_NODES // 5, D_FEAT), lambda i: (0, i, 0)),
        ],
        out_specs=pl.BlockSpec((N_NODES // 5, D_FEAT), lambda i: (i, 0)),
    )(partials)


def kernel(x, edge_index):
    ei6 = edge_index.reshape(2, NC, NS, NGROUP, GCHUNK, CHUNK)
    partials = _propagate(x, ei6)
    return _combine(partials)
